# baseline (speedup 1.0000x reference)
import jax
import jax.numpy as jnp
from jax.experimental import pallas as pl
from jax.experimental.pallas import tpu as pltpu


def _fused_block_kernel(xp_ref, w_ref, be_ref, w1_ref, b1_ref, w2_ref,
                        b2_ref, g_ref, o_ref):
    B, H, W, C = o_ref.shape
    K = w_ref.shape[0]
    P = K // 2

    for i in range(B):
        acc = jnp.broadcast_to(be_ref[...].reshape(1, 1, C), (H, W, C))
        for kh in range(K):
            for kw in range(K):
                win = xp_ref[i, kh:kh + H, kw:kw + W, :]
                wv = w_ref[kh, kw:kw + 1, :].reshape(1, 1, C)
                acc = acc + win * wv

        t = acc.reshape(H * W, C).astype(jnp.bfloat16)
        h = jnp.dot(t, w1_ref[...], preferred_element_type=jnp.float32)
        h = h + b1_ref[...]
        h = h * jnp.clip(h + 3.0, 0.0, 6.0)
        y = jnp.dot(h.astype(jnp.bfloat16), w2_ref[...],
                    preferred_element_type=jnp.float32)
        y = y + b2_ref[...]

        xres = xp_ref[i, P:P + H, P:P + W, :]
        out = xres + g_ref[...].reshape(1, 1, C) * y.reshape(H, W, C)
        o_ref[i] = out.astype(o_ref.dtype)


def kernel(x, w_dw, b_dw, bn_w, bn_b, bn_mean, bn_var, w1, b1, w2, b2, gamma):
    N, C, H, W = x.shape
    K = w_dw.shape[-1]
    P = K // 2
    CE = w1.shape[1]
    Hp, Wp = H + 2 * P, W + 2 * P

    s = bn_w * jax.lax.rsqrt(bn_var + 1e-5)
    w_eff = jnp.transpose(w_dw[:, 0, :, :], (1, 2, 0)) * s
    b_eff = ((b_dw - bn_mean) * s + bn_b).reshape(1, C)

    x_nhwc = jnp.transpose(x, (0, 2, 3, 1))
    x_pad = jnp.pad(x_nhwc, ((0, 0), (P, P), (P, P), (0, 0)))

    B = 2 if N % 2 == 0 else 1
    out_nhwc = pl.pallas_call(
        _fused_block_kernel,
        out_shape=jax.ShapeDtypeStruct((N, H, W, C), x.dtype),
        grid=(N // B,),
        in_specs=[
            pl.BlockSpec((B, Hp, Wp, C), lambda n: (n, 0, 0, 0)),
            pl.BlockSpec((K, K, C), lambda n: (0, 0, 0)),
            pl.BlockSpec((1, C), lambda n: (0, 0)),
            pl.BlockSpec((C, CE), lambda n: (0, 0)),
            pl.BlockSpec((1, CE), lambda n: (0, 0)),
            pl.BlockSpec((CE, C), lambda n: (0, 0)),
            pl.BlockSpec((1, C), lambda n: (0, 0)),
            pl.BlockSpec((1, C), lambda n: (0, 0)),
        ],
        out_specs=pl.BlockSpec((B, H, W, C), lambda n: (n, 0, 0, 0)),
        compiler_params=pltpu.CompilerParams(dimension_semantics=("parallel",)),
    )(x_pad, w_eff, b_eff, w1.astype(jnp.bfloat16), b1.reshape(1, CE),
      (w2 * (1.0 / 6.0)).astype(jnp.bfloat16), b2.reshape(1, C),
      gamma.reshape(1, C))

    return jnp.transpose(out_nhwc, (0, 3, 1, 2))
